# baseline (speedup 1.0000x reference)
import functools

import jax
import jax.numpy as jnp
from jax import lax
from jax.experimental import pallas as pl
from jax.experimental.layout import Format, Layout
from jax.experimental.pallas import tpu as pltpu
from jax.experimental.pallas import tpu_sc as plsc

NUM_BUCKETS = 32
NUM_HEADS = 12
QL = 2048
KL = 2048

NC = 2
NS = 16
LANES = 16
NW = NC * NS
TOTAL_ROWS = NUM_HEADS * QL
ROWS_PER_W = TOTAL_ROWS // NW
NSHIFT = 8
VD_SH = 4096
VD_STEPS = VD_SH // LANES
INFLIGHT = 8

_THRESHOLDS = (12, 16, 23, 32, 46, 64, 91)


def _body(bias_hbm, out_hbm, bias_v, vd_v, sem):
  wid = lax.axis_index("s") * NC + lax.axis_index("c")
  row0 = wid * ROWS_PER_W
  h0 = lax.shift_right_logical(row0, 11)

  pltpu.sync_copy(bias_hbm, bias_v)

  def compute_vd(hh, s):
    h = jnp.minimum(h0 + hh, NUM_HEADS - 1)
    head_idx = jnp.full((LANES,), h, dtype=jnp.int32)
    base = (hh * NSHIFT + s) * VD_SH

    def step(t, carry):
      d = t * LANES + lax.iota(jnp.int32, LANES) + s
      rel = d - (QL - 1)
      n = jnp.abs(rel)
      large = jnp.full((LANES,), 8, dtype=jnp.int32)
      for thr in _THRESHOLDS:
        large = large + jnp.where(n >= thr, 1, 0).astype(jnp.int32)
      bucket = jnp.where(n < 8, n, large) + jnp.where(rel > 0, 16, 0)
      vals = plsc.load_gather(bias_v, [bucket, head_idx])
      vd_v[pl.ds(base + t * LANES, LANES)] = vals
      return carry

    lax.fori_loop(0, VD_STEPS, step, 0)

  for hh in range(2):
    for s in range(NSHIFT):
      compute_vd(hh, s)

  def row_group(g, carry):
    r0 = row0 + g * INFLIGHT
    copies = []
    for b in range(INFLIGHT):
      r = r0 + b
      h = lax.shift_right_logical(r, 11)
      i = lax.bitwise_and(r, QL - 1)
      hh = h - h0
      off = (QL - 1) - i
      s = lax.bitwise_and(off, NSHIFT - 1)
      src_base = (hh * NSHIFT + s) * VD_SH + lax.bitwise_and(
          off, ~(NSHIFT - 1)
      )
      src_base = pl.multiple_of(src_base, NSHIFT)
      copies.append(
          pltpu.async_copy(
              vd_v.at[pl.ds(src_base, KL)], out_hbm.at[h, i], sem
          )
      )
    for c in copies:
      c.wait()
    return carry

  lax.fori_loop(0, ROWS_PER_W // INFLIGHT, row_group, 0)


_sc_kernel = pl.kernel(
    _body,
    out_type=jax.ShapeDtypeStruct((NUM_HEADS, QL, KL), jnp.float32),
    mesh=plsc.VectorSubcoreMesh(core_axis_name="c", subcore_axis_name="s"),
    compiler_params=pltpu.CompilerParams(
        needs_layout_passes=False, use_tc_tiling_on_sc=False
    ),
    scratch_types=[
        pltpu.VMEM((NUM_BUCKETS, NUM_HEADS), jnp.float32),
        pltpu.VMEM((2 * NSHIFT * VD_SH,), jnp.float32),
        pltpu.SemaphoreType.DMA,
    ],
)


def _impl(query_length, key_length, relative_attention_bias):
  del query_length, key_length
  return _sc_kernel(relative_attention_bias)


_impl.__name__ = "kernel"
_jitted = None


def kernel(query_length, key_length, relative_attention_bias):
  global _jitted
  if _jitted is None:
    fmt = Format(
        Layout(major_to_minor=(0, 1, 2), tiling=()),
        jax.sharding.SingleDeviceSharding(jax.devices()[0]),
    )
    _jitted = jax.jit(_impl, out_shardings=fmt)
  return _jitted(query_length, key_length, relative_attention_bias)


# speedup vs baseline: 1.0023x; 1.0023x over previous
import functools

import jax
import jax.numpy as jnp
from jax import lax
from jax.experimental import pallas as pl
from jax.experimental.layout import Format, Layout
from jax.experimental.pallas import tpu as pltpu
from jax.experimental.pallas import tpu_sc as plsc

NUM_BUCKETS = 32
NUM_HEADS = 12
QL = 2048
KL = 2048

NC = 2
NS = 16
LANES = 16
NW = NC * NS
TOTAL_ROWS = NUM_HEADS * QL
ROWS_PER_W = TOTAL_ROWS // NW
NSHIFT = 8
VD_SH = 4096
VD_STEPS = VD_SH // LANES
INFLIGHT = 8

_THRESHOLDS = (12, 16, 23, 32, 46, 64, 91)


def _body(bias_hbm, out_hbm, bias_v, vd_v, sem):
  wid = lax.axis_index("s") * NC + lax.axis_index("c")
  row0 = wid * ROWS_PER_W
  h0 = lax.shift_right_logical(row0, 11)

  pltpu.sync_copy(bias_hbm, bias_v)

  def compute_vd(hh, s):
    h = jnp.minimum(h0 + hh, NUM_HEADS - 1)
    head_idx = jnp.full((LANES,), h, dtype=jnp.int32)
    base = (hh * NSHIFT + s) * VD_SH

    def step(t, carry):
      d = t * LANES + lax.iota(jnp.int32, LANES) + s
      rel = d - (QL - 1)
      n = jnp.abs(rel)
      large = jnp.full((LANES,), 8, dtype=jnp.int32)
      for thr in _THRESHOLDS:
        large = large + jnp.where(n >= thr, 1, 0).astype(jnp.int32)
      bucket = jnp.where(n < 8, n, large) + jnp.where(rel > 0, 16, 0)
      vals = plsc.load_gather(bias_v, [bucket, head_idx])
      vd_v[pl.ds(base + t * LANES, LANES)] = vals
      return carry

    lax.fori_loop(0, VD_STEPS, step, 0)

  for hh in range(2):
    for s in range(NSHIFT):
      compute_vd(hh, s)

  def row_group(g, carry):
    r0 = row0 + g * INFLIGHT
    copies = []
    for b in range(INFLIGHT):
      r = r0 + b
      h = lax.shift_right_logical(r, 11)
      i = lax.bitwise_and(r, QL - 1)
      hh = h - h0
      off = (QL - 1) - i
      s = lax.bitwise_and(off, NSHIFT - 1)
      src_base = (hh * NSHIFT + s) * VD_SH + lax.bitwise_and(
          off, ~(NSHIFT - 1)
      )
      src_base = pl.multiple_of(src_base, NSHIFT)
      copies.append(
          pltpu.async_copy(
              vd_v.at[pl.ds(src_base, KL)], out_hbm.at[h, i], sem
          )
      )
    for c in copies:
      c.wait()
    return carry

  lax.fori_loop(0, ROWS_PER_W // INFLIGHT, row_group, 0)


_sc_kernel = pl.kernel(
    _body,
    out_type=jax.ShapeDtypeStruct((NUM_HEADS, QL, KL), jnp.float32),
    mesh=plsc.VectorSubcoreMesh(core_axis_name="c", subcore_axis_name="s"),
    compiler_params=pltpu.CompilerParams(
        needs_layout_passes=False, use_tc_tiling_on_sc=False
    ),
    scratch_types=[
        pltpu.VMEM((NUM_BUCKETS, NUM_HEADS), jnp.float32),
        pltpu.VMEM((2 * NSHIFT * VD_SH,), jnp.float32),
        pltpu.SemaphoreType.DMA,
    ],
)


def _impl(query_length, key_length, relative_attention_bias):
  del query_length, key_length
  return _sc_kernel(relative_attention_bias)


_impl.__name__ = "kernel"
_jitted = None


def kernel(query_length, key_length, relative_attention_bias):
  global _jitted
  if _jitted is None:
    fmt = Format(
        Layout(major_to_minor=(0, 1, 2), tiling=((8,),)),
        jax.sharding.SingleDeviceSharding(jax.devices()[0]),
    )
    _jitted = jax.jit(_impl, out_shardings=fmt)
  return _jitted(query_length, key_length, relative_attention_bias)


# speedup vs baseline: 1.0394x; 1.0370x over previous
import jax
import jax.numpy as jnp
from jax import lax
from jax.experimental import pallas as pl
from jax.experimental.pallas import tpu as pltpu
from jax.experimental.pallas import tpu_sc as plsc

NUM_BUCKETS = 32
NUM_HEADS = 12
QL = 2048
KL = 2048

NC = 2
NS = 16
LANES = 16
NW = NC * NS
TOTAL_ROWS = NUM_HEADS * QL
ROWS_PER_W = TOTAL_ROWS // NW
NSHIFT = 8
VD_SH = 4096
VD_STEPS = VD_SH // LANES
INFLIGHT = 8

_THRESHOLDS = (12, 16, 23, 32, 46, 64, 91)


def _body(bias_hbm, out_hbm, bias_v, vd_v, sem):
  wid = lax.axis_index("s") * NC + lax.axis_index("c")
  row0 = wid * ROWS_PER_W
  h0 = lax.shift_right_logical(row0, 11)

  pltpu.sync_copy(bias_hbm, bias_v)

  def compute_vd(hh, s):
    h = jnp.minimum(h0 + hh, NUM_HEADS - 1)
    head_idx = jnp.full((LANES,), h, dtype=jnp.int32)
    base = (hh * NSHIFT + s) * VD_SH

    def step(t, carry):
      d = t * LANES + lax.iota(jnp.int32, LANES) + s
      rel = d - (QL - 1)
      n = jnp.abs(rel)
      large = jnp.full((LANES,), 8, dtype=jnp.int32)
      for thr in _THRESHOLDS:
        large = large + jnp.where(n >= thr, 1, 0).astype(jnp.int32)
      bucket = jnp.where(n < 8, n, large) + jnp.where(rel > 0, 16, 0)
      vals = plsc.load_gather(bias_v, [bucket, head_idx])
      vd_v[pl.ds(base + t * LANES, LANES)] = vals
      return carry

    lax.fori_loop(0, VD_STEPS, step, 0)

  i_lo0 = lax.bitwise_and(row0, QL - 1)
  i_hi0 = jnp.minimum(i_lo0 + ROWS_PER_W, QL)
  i_hi1 = row0 + ROWS_PER_W - ((h0 + 1) * QL)

  def fire_group(hh, s, h):
    i_lo = i_lo0 * (1 - hh)
    i_hi = jnp.where(hh == 0, i_hi0, jnp.maximum(i_hi1, 0))
    off_lo = (QL - 1) - (i_hi - 1)
    off_hi = (QL - 1) - i_lo
    q_lo = lax.shift_right_logical(off_lo - s + 7, 3)
    q_hi = lax.shift_right_logical(off_hi - s, 3)
    base = (hh * NSHIFT + s) * VD_SH

    def fire(q, carry):
      off = q * NSHIFT + s
      i = (QL - 1) - off
      src_base = pl.multiple_of(base + q * NSHIFT, NSHIFT)
      pltpu.async_copy(vd_v.at[pl.ds(src_base, KL)], out_hbm.at[h, i], sem)
      return carry

    lax.fori_loop(q_lo, q_hi + 1, fire, 0)
    return jnp.maximum(q_hi + 1 - q_lo, 0)

  def drain(count):
    def one(_, carry):
      pltpu.make_async_copy(
          vd_v.at[pl.ds(0, KL)], out_hbm.at[0, 0], sem
      ).wait()
      return carry

    lax.fori_loop(0, count, one, 0)

  prev = None
  for hh in range(2):
    h = jnp.minimum(h0 + hh, NUM_HEADS - 1)
    for s in range(NSHIFT):
      compute_vd(hh, s)
      if prev is not None:
        drain(prev)
      prev = fire_group(hh, s, h)
  drain(prev)


_sc_kernel = pl.kernel(
    _body,
    out_type=jax.ShapeDtypeStruct((NUM_HEADS, QL, KL), jnp.float32),
    mesh=plsc.VectorSubcoreMesh(core_axis_name="c", subcore_axis_name="s"),
    compiler_params=pltpu.CompilerParams(
        needs_layout_passes=False, use_tc_tiling_on_sc=False
    ),
    scratch_types=[
        pltpu.VMEM((NUM_BUCKETS, NUM_HEADS), jnp.float32),
        pltpu.VMEM((2 * NSHIFT * VD_SH,), jnp.float32),
        pltpu.SemaphoreType.DMA,
    ],
)


@jax.jit
def kernel(query_length, key_length, relative_attention_bias):
  del query_length, key_length
  return _sc_kernel(relative_attention_bias)


# speedup vs baseline: 1.8129x; 1.7443x over previous
import jax
import jax.numpy as jnp
from jax import lax
from jax.experimental import pallas as pl
from jax.experimental.pallas import tpu as pltpu
from jax.experimental.pallas import tpu_sc as plsc

NUM_BUCKETS = 32
NUM_HEADS = 12
QL = 2048
KL = 2048

NC = 2
NS = 16
LANES = 16
NW = NC * NS
NSHIFT = 8
BAND_LO = 1664
BAND_W = 768
BAND_STEPS = BAND_W // LANES
N_UNITS = NUM_HEADS * NSHIFT

_THRESHOLDS = (12, 16, 23, 32, 46, 64, 91)


def _bucket_of(d):
  rel = d - (QL - 1)
  n = jnp.abs(rel)
  large = jnp.full((LANES,), 8, dtype=jnp.int32)
  for thr in _THRESHOLDS:
    large = large + jnp.where(n >= thr, 1, 0).astype(jnp.int32)
  return jnp.where(n < 8, n, large) + jnp.where(rel > 0, 16, 0)


def _band_body(bias_hbm, r8_hbm, bias_v, buf_v):
  wid = lax.axis_index("s") * NC + lax.axis_index("c")
  pltpu.sync_copy(bias_hbm, bias_v)

  for j in range(N_UNITS // NW):
    u = wid + NW * j
    h = lax.shift_right_logical(u, 3)
    k = lax.bitwise_and(u, NSHIFT - 1)
    head_idx = jnp.full((LANES,), h, dtype=jnp.int32)

    def step(t, carry, k=k, head_idx=head_idx):
      d = BAND_LO + 7 - k + t * LANES + lax.iota(jnp.int32, LANES)
      vals = plsc.load_gather(bias_v, [_bucket_of(d), head_idx])
      buf_v[pl.ds(t * LANES, LANES)] = vals
      return carry

    lax.fori_loop(0, BAND_STEPS, step, 0)
    pltpu.sync_copy(buf_v, r8_hbm.at[h, k])


_sc_band = pl.kernel(
    _band_body,
    out_type=jax.ShapeDtypeStruct((NUM_HEADS, NSHIFT, BAND_W), jnp.float32),
    mesh=plsc.VectorSubcoreMesh(core_axis_name="c", subcore_axis_name="s"),
    compiler_params=pltpu.CompilerParams(
        needs_layout_passes=False, use_tc_tiling_on_sc=False
    ),
    scratch_types=[
        pltpu.VMEM((NUM_BUCKETS, NUM_HEADS), jnp.float32),
        pltpu.VMEM((BAND_W,), jnp.float32),
    ],
)

_BROWS = 128
_GRID_G = QL // _BROWS


def _tc_body(bias_s, r8_ref, out_ref):
  h = pl.program_id(0)
  g = pl.program_id(1)
  cn = bias_s[15, h]
  cp = bias_s[31, h]
  for ct in range(16):
    cval = jnp.where(ct < g, cn, cp)
    out_ref[0, :, 128 * ct:128 * (ct + 1)] = jnp.full(
        (_BROWS, 128), cval, jnp.float32
    )
  for dd in (-1, 0, 1):
    cond = jnp.logical_and(g + dd >= 0, g + dd <= _GRID_G - 1)

    @pl.when(cond)
    def _(dd=dd):
      pieces = [
          r8_ref[0, :, 376 + 128 * dd - 8 * a:504 + 128 * dd - 8 * a]
          for a in range(16)
      ]
      w = jnp.concatenate(pieces, axis=0)
      out_ref[0, :, pl.ds((g + dd) * 128, 128)] = w


_tc_fill = pl.pallas_call(
    _tc_body,
    grid=(NUM_HEADS, _GRID_G),
    in_specs=[
        pl.BlockSpec(memory_space=pltpu.SMEM),
        pl.BlockSpec((1, NSHIFT, BAND_W), lambda h, g: (h, 0, 0)),
    ],
    out_specs=pl.BlockSpec((1, _BROWS, KL), lambda h, g: (h, g, 0)),
    out_shape=jax.ShapeDtypeStruct((NUM_HEADS, QL, KL), jnp.float32),
    compiler_params=pltpu.CompilerParams(
        dimension_semantics=("arbitrary", "arbitrary")
    ),
)


@jax.jit
def kernel(query_length, key_length, relative_attention_bias):
  del query_length, key_length
  r8 = _sc_band(relative_attention_bias)
  return _tc_fill(relative_attention_bias, r8)
